# baseline (speedup 1.0000x reference)
import functools

import jax
import jax.numpy as jnp
from jax import lax
from jax.experimental import pallas as pl
from jax.experimental.pallas import tpu as pltpu
from jax.experimental.pallas import tpu_sc as plsc

N_NODES = 10000
N_EDGES = 320000
D_FEAT = 128
LANES = 16

NUM_CORES = 2
NUM_SUBCORES = 16
NUM_WORKERS = NUM_CORES * NUM_SUBCORES
BLK = 128
NBLK_TOT = N_EDGES // BLK
NBLK = NBLK_TOT // NUM_WORKERS
NBLK_EXTRA = NBLK_TOT - NBLK * NUM_WORKERS
N_PAD = 10240
ROWS_PER_TILE = N_PAD // NUM_SUBCORES
ZROWS = BLK

_mesh = plsc.VectorSubcoreMesh(core_axis_name="c", subcore_axis_name="s")


@functools.partial(
    pl.kernel,
    mesh=_mesh,
    compiler_params=pltpu.CompilerParams(use_tc_tiling_on_sc=False),
    out_type=jax.ShapeDtypeStruct((NUM_CORES, N_PAD, D_FEAT), jnp.float32),
    scratch_types=[
        pltpu.VMEM((2, BLK), jnp.int32),
        pltpu.VMEM((2, BLK), jnp.int32),
        pltpu.VMEM((2, BLK), jnp.float32),
        pltpu.VMEM((2, BLK, D_FEAT // 2), jnp.int32),
        pltpu.VMEM((BLK, D_FEAT), jnp.float32),
        pltpu.VMEM_SHARED((N_PAD, D_FEAT), jnp.float32),
        pltpu.SemaphoreType.DMA((2,)),
        pltpu.SemaphoreType.DMA((2,)),
    ],
)
def _spmm(x_hbm, col_hbm, row_hbm, w_hbm, out_hbm,
          colv, rowv, wv, rows, scaled, acc, sem_i, sem_g):
    c = lax.axis_index("c")
    s = lax.axis_index("s")
    wid = s * NUM_CORES + c

    def _zfill(t, carry):
        i = t // (D_FEAT // LANES)
        j = t % (D_FEAT // LANES)
        scaled[i, pl.ds(j * LANES, LANES)] = jnp.zeros((LANES,), jnp.float32)
        return carry

    lax.fori_loop(0, ZROWS * (D_FEAT // LANES), _zfill, 0)

    def _zacc(i, carry):
        pltpu.sync_copy(scaled,
                        acc.at[pl.ds(s * ROWS_PER_TILE + i * ZROWS, ZROWS)])
        return carry

    lax.fori_loop(0, ROWS_PER_TILE // ZROWS, _zacc, 0)
    plsc.subcore_barrier()

    blk0 = wid * NBLK

    def _issue_idx(b, p):
        off = (blk0 + b) * BLK
        pltpu.async_copy(col_hbm.at[pl.ds(off, BLK)], colv.at[p], sem_i.at[p])
        pltpu.async_copy(row_hbm.at[pl.ds(off, BLK)], rowv.at[p], sem_i.at[p])
        pltpu.async_copy(w_hbm.at[pl.ds(off, BLK)], wv.at[p], sem_i.at[p])

    def _wait_idx(p):
        pltpu.make_async_copy(col_hbm.at[pl.ds(0, BLK)], colv.at[p],
                              sem_i.at[p]).wait()
        pltpu.make_async_copy(row_hbm.at[pl.ds(0, BLK)], rowv.at[p],
                              sem_i.at[p]).wait()
        pltpu.make_async_copy(w_hbm.at[pl.ds(0, BLK)], wv.at[p],
                              sem_i.at[p]).wait()

    def _issue_gather(p):
        pltpu.async_copy(x_hbm.at[colv.at[p]], rows.at[p], sem_g.at[p])

    def _wait_gather(p):
        pltpu.make_async_copy(x_hbm.at[colv.at[p]], rows.at[p],
                              sem_g.at[p]).wait()

    gdims = lax.GatherDimensionNumbers(
        offset_dims=(), collapsed_slice_dims=(0,), start_index_map=(0,))
    _HIMASK = jnp.full((LANES,), -65536, jnp.int32)

    def _scale(p):
        def _grp(g, carry):
            wreg = wv[p, pl.ds(g * LANES, LANES)]
            for e in range(LANES):
                wvec = lax.gather(
                    wreg, jnp.full((LANES, 1), e, jnp.int32), gdims,
                    slice_sizes=(1,),
                    mode=lax.GatherScatterMode.PROMISE_IN_BOUNDS)
                r = g * LANES + e
                for j in range(D_FEAT // (2 * LANES)):
                    vi = rows[p, r, pl.ds(j * LANES, LANES)]
                    fe = lax.bitcast_convert_type(
                        lax.shift_left(vi, 16), jnp.float32)
                    fo = lax.bitcast_convert_type(vi & _HIMASK, jnp.float32)
                    scaled[r, pl.ds(j * 2 * LANES, LANES)] = fe * wvec
                    scaled[r, pl.ds(j * 2 * LANES + LANES, LANES)] = fo * wvec
            return carry

        lax.fori_loop(0, BLK // LANES, _grp, 0)

    def _section(b, p, q):
        _wait_gather(p)
        _scale(p)

        @pl.when(b + 1 < NBLK)
        def _():
            _wait_idx(q)
            _issue_gather(q)

        pltpu.sync_copy(scaled, acc.at[rowv.at[p]], add=True)

        @pl.when(b + 2 < NBLK)
        def _():
            _issue_idx(b + 2, p)

    _issue_idx(0, 0)
    _wait_idx(0)
    _issue_gather(0)
    _issue_idx(1, 1)

    def _body(i2, carry):
        _section(2 * i2, 0, 1)
        _section(2 * i2 + 1, 1, 0)
        return carry

    lax.fori_loop(0, NBLK // 2, _body, 0)

    @pl.when(wid < NBLK_EXTRA)
    def _():
        eb = NBLK_TOT - NBLK_EXTRA + wid - blk0
        _issue_idx(eb, 0)
        _wait_idx(0)
        _issue_gather(0)
        _wait_gather(0)
        _scale(0)
        pltpu.sync_copy(scaled, acc.at[rowv.at[0]], add=True)

    plsc.subcore_barrier()

    pltpu.sync_copy(acc.at[pl.ds(s * ROWS_PER_TILE, ROWS_PER_TILE)],
                    out_hbm.at[c, pl.ds(s * ROWS_PER_TILE, ROWS_PER_TILE)])


_ADD_BS = 512


def _combine_body(p_ref, o_ref, ob_ref):
    t = p_ref[0] + p_ref[1]
    t = t.reshape(_ADD_BS, D_FEAT // 32, 2, 16)
    t = jnp.transpose(t, (0, 1, 3, 2)).reshape(_ADD_BS, D_FEAT)
    o_ref[...] = t
    ob_ref[...] = t.astype(jnp.bfloat16)


_combine = pl.pallas_call(
    _combine_body,
    grid=(N_PAD // _ADD_BS,),
    in_specs=[pl.BlockSpec((2, _ADD_BS, D_FEAT), lambda i: (0, i, 0))],
    out_specs=[pl.BlockSpec((_ADD_BS, D_FEAT), lambda i: (i, 0)),
               pl.BlockSpec((_ADD_BS, D_FEAT), lambda i: (i, 0))],
    out_shape=[jax.ShapeDtypeStruct((N_PAD, D_FEAT), jnp.float32),
               jax.ShapeDtypeStruct((N_PAD, D_FEAT), jnp.bfloat16)],
)


def _pack_pairs(xb):
    return lax.bitcast_convert_type(
        xb.reshape(N_PAD, D_FEAT // 2, 2), jnp.int32)


def kernel(features, edge_index, edge_weight, degree):
    row = edge_index[0].astype(jnp.int32)
    col = edge_index[1].astype(jnp.int32)
    w = edge_weight.astype(jnp.float32)
    xf = jnp.pad(features, ((0, N_PAD - N_NODES), (0, 0)))
    xb = _pack_pairs(xf.astype(jnp.bfloat16))

    def body(_, carry):
        _, x_packed = carry
        partial = _spmm(x_packed, col, row, w)
        o_f32, o_bf16 = _combine(partial)
        return (o_f32, _pack_pairs(o_bf16))

    out_f32, _ = lax.fori_loop(0, degree, body, (xf, xb))
    return out_f32[:N_NODES]


# speedup vs baseline: 2.8070x; 2.8070x over previous
import functools

import jax
import jax.numpy as jnp
from jax import lax
from jax.experimental import pallas as pl
from jax.experimental.pallas import tpu as pltpu
from jax.experimental.pallas import tpu_sc as plsc

N_NODES = 10000
N_EDGES = 320000
D_FEAT = 128
LANES = 16

NUM_CORES = 2
NUM_SUBCORES = 16
NUM_WORKERS = NUM_CORES * NUM_SUBCORES
BLK = 128
NBLK_TOT = N_EDGES // BLK
NBLK = NBLK_TOT // NUM_WORKERS
NBLK_EXTRA = NBLK_TOT - NBLK * NUM_WORKERS
N_PAD = 10240
ROWS_PER_TILE = N_PAD // NUM_SUBCORES
ZROWS = BLK

_mesh = plsc.VectorSubcoreMesh(core_axis_name="c", subcore_axis_name="s")


@functools.partial(
    pl.kernel,
    mesh=_mesh,
    out_type=jax.ShapeDtypeStruct((NUM_CORES, N_PAD, D_FEAT), jnp.float32),
    scratch_types=[
        pltpu.VMEM((2, BLK), jnp.int32),
        pltpu.VMEM((4, BLK), jnp.int32),
        pltpu.VMEM((2, BLK), jnp.float32),
        pltpu.VMEM((2, BLK, D_FEAT), jnp.float32),
        pltpu.VMEM_SHARED((N_PAD, D_FEAT), jnp.float32),
        pltpu.SemaphoreType.DMA((2,)),
        pltpu.SemaphoreType.DMA((2,)),
        pltpu.SemaphoreType.DMA((4,)),
    ],
)
def _spmm(x_hbm, col_hbm, row_hbm, w_hbm, out_hbm,
          colv, rowv, wv, rows, acc, sem_i, sem_g, sem_s):
    c = lax.axis_index("c")
    s = lax.axis_index("s")
    wid = s * NUM_CORES + c

    def _zfill(t, carry):
        i = t // (D_FEAT // LANES)
        j = t % (D_FEAT // LANES)
        rows[0, i, pl.ds(j * LANES, LANES)] = jnp.zeros((LANES,), jnp.float32)
        return carry

    lax.fori_loop(0, ZROWS * (D_FEAT // LANES), _zfill, 0)

    blk0 = wid * NBLK

    def _issue_idx(b, p, m):
        off = (blk0 + b) * BLK
        pltpu.async_copy(col_hbm.at[pl.ds(off, BLK)], colv.at[p], sem_i.at[p])
        pltpu.async_copy(row_hbm.at[pl.ds(off, BLK)], rowv.at[m], sem_i.at[p])
        pltpu.async_copy(w_hbm.at[pl.ds(off, BLK)], wv.at[p], sem_i.at[p])

    def _wait_idx(p):
        pltpu.make_async_copy(col_hbm.at[pl.ds(0, BLK)], colv.at[p],
                              sem_i.at[p]).wait()
        pltpu.make_async_copy(row_hbm.at[pl.ds(0, BLK)], rowv.at[0],
                              sem_i.at[p]).wait()
        pltpu.make_async_copy(w_hbm.at[pl.ds(0, BLK)], wv.at[p],
                              sem_i.at[p]).wait()

    def _issue_scatter(p, m):
        pltpu.async_copy(rows.at[p], acc.at[rowv.at[m]], sem_s.at[m], add=True)

    def _wait_scatter(m):
        pltpu.make_async_copy(rows.at[0], acc.at[rowv.at[m]],
                              sem_s.at[m]).wait()

    def _issue_gather(p):
        pltpu.async_copy(x_hbm.at[colv.at[p]], rows.at[p], sem_g.at[p])

    def _wait_gather(p):
        pltpu.make_async_copy(x_hbm.at[colv.at[p]], rows.at[p],
                              sem_g.at[p]).wait()

    gdims = lax.GatherDimensionNumbers(
        offset_dims=(), collapsed_slice_dims=(0,), start_index_map=(0,))

    def _scale(p):
        def _grp(g, carry):
            wreg = wv[p, pl.ds(g * LANES, LANES)]
            for e in range(LANES):
                wvec = lax.gather(
                    wreg, jnp.full((LANES, 1), e, jnp.int32), gdims,
                    slice_sizes=(1,),
                    mode=lax.GatherScatterMode.PROMISE_IN_BOUNDS)
                r = g * LANES + e
                for j in range(D_FEAT // LANES):
                    sl = pl.ds(j * LANES, LANES)
                    rows[p, r, sl] = rows[p, r, sl] * wvec
            return carry

        lax.fori_loop(0, BLK // LANES, _grp, 0)

    def _section(b, p, q):
        m = lax.rem(b, 4)
        _wait_gather(p)
        _scale(p)

        @pl.when(b + 1 < NBLK)
        def _():
            @pl.when(b >= 1)
            def _():
                _wait_scatter(lax.rem(b + 3, 4))

            _wait_idx(q)
            _issue_gather(q)

        _issue_scatter(p, m)

        @pl.when(b + 2 < NBLK)
        def _():
            _issue_idx(b + 2, p, lax.rem(b + 2, 4))

    _issue_idx(0, 0, 0)
    _issue_idx(1, 1, 1)

    def _zacc(i, carry):
        pltpu.sync_copy(rows.at[0],
                        acc.at[pl.ds(s * ROWS_PER_TILE + i * ZROWS, ZROWS)])
        return carry

    lax.fori_loop(0, ROWS_PER_TILE // ZROWS, _zacc, 0)
    _wait_idx(0)
    _issue_gather(0)
    plsc.subcore_barrier()

    def _body(i2, carry):
        _section(2 * i2, 0, 1)
        _section(2 * i2 + 1, 1, 0)
        return carry

    lax.fori_loop(0, NBLK // 2, _body, 0)

    _wait_scatter((NBLK - 2) % 4)
    _wait_scatter((NBLK - 1) % 4)

    @pl.when(wid < NBLK_EXTRA)
    def _():
        eb = NBLK_TOT - NBLK_EXTRA + wid - blk0
        _issue_idx(eb, 0, 0)
        _wait_idx(0)
        _issue_gather(0)
        _wait_gather(0)
        _scale(0)
        pltpu.sync_copy(rows.at[0], acc.at[rowv.at[0]], add=True)

    plsc.subcore_barrier()

    pltpu.sync_copy(acc.at[pl.ds(s * ROWS_PER_TILE, ROWS_PER_TILE)],
                    out_hbm.at[c, pl.ds(s * ROWS_PER_TILE, ROWS_PER_TILE)])


_ADD_BS = 512


def _add_body(p_ref, o_ref):
    o_ref[...] = p_ref[0] + p_ref[1]


_combine = pl.pallas_call(
    _add_body,
    grid=(N_PAD // _ADD_BS,),
    in_specs=[pl.BlockSpec((2, _ADD_BS, D_FEAT), lambda i: (0, i, 0))],
    out_specs=pl.BlockSpec((_ADD_BS, D_FEAT), lambda i: (i, 0)),
    out_shape=jax.ShapeDtypeStruct((N_PAD, D_FEAT), jnp.float32),
)


def kernel(features, edge_index, edge_weight, degree):
    row = edge_index[0].astype(jnp.int32)
    col = edge_index[1].astype(jnp.int32)
    w = edge_weight.astype(jnp.float32)
    x0 = jnp.pad(features, ((0, N_PAD - N_NODES), (0, 0)))

    def body(_, x):
        partial = _spmm(x, col, row, w)
        return _combine(partial)

    out = lax.fori_loop(0, degree, body, x0)
    return out[:N_NODES]


# speedup vs baseline: 2.8598x; 1.0188x over previous
import functools

import jax
import jax.numpy as jnp
from jax import lax
from jax.experimental import pallas as pl
from jax.experimental.pallas import tpu as pltpu
from jax.experimental.pallas import tpu_sc as plsc

N_NODES = 10000
N_EDGES = 320000
D_FEAT = 128
LANES = 16

NUM_CORES = 2
NUM_SUBCORES = 16
NUM_WORKERS = NUM_CORES * NUM_SUBCORES
BLK = 128
NBLK_TOT = N_EDGES // BLK
NBLK = NBLK_TOT // NUM_WORKERS
NBLK_EXTRA = NBLK_TOT - NBLK * NUM_WORKERS
N_PAD = 10240
ROWS_PER_TILE = N_PAD // NUM_SUBCORES
ZROWS = BLK

_mesh = plsc.VectorSubcoreMesh(core_axis_name="c", subcore_axis_name="s")


@functools.partial(
    pl.kernel,
    mesh=_mesh,
    out_type=jax.ShapeDtypeStruct((NUM_CORES, N_PAD, D_FEAT), jnp.float32),
    scratch_types=[
        pltpu.VMEM((2, 2, BLK), jnp.int32),
        pltpu.VMEM((2, 1, BLK), jnp.float32),
        pltpu.VMEM((2, BLK, D_FEAT), jnp.float32),
        pltpu.VMEM_SHARED((N_PAD, D_FEAT), jnp.float32),
        pltpu.SemaphoreType.DMA((2,)),
        pltpu.SemaphoreType.DMA((2,)),
    ],
)
def _spmm(x_hbm, idx_hbm, w_hbm, out_hbm,
          ibuf, wbuf, rows, acc, sem_i, sem_g):
    c = lax.axis_index("c")
    s = lax.axis_index("s")
    wid = s * NUM_CORES + c

    def _zfill(t, carry):
        i = t // (D_FEAT // LANES)
        j = t % (D_FEAT // LANES)
        rows[0, i, pl.ds(j * LANES, LANES)] = jnp.zeros((LANES,), jnp.float32)
        return carry

    lax.fori_loop(0, ZROWS * (D_FEAT // LANES), _zfill, 0)

    blk0 = wid * NBLK

    def _issue_idx(b, p):
        blk = blk0 + b
        pltpu.async_copy(idx_hbm.at[blk], ibuf.at[p], sem_i.at[p])
        pltpu.async_copy(w_hbm.at[blk], wbuf.at[p], sem_i.at[p])

    def _wait_idx(p):
        pltpu.make_async_copy(idx_hbm.at[0], ibuf.at[p], sem_i.at[p]).wait()
        pltpu.make_async_copy(w_hbm.at[0], wbuf.at[p], sem_i.at[p]).wait()

    def _issue_gather(p):
        pltpu.async_copy(x_hbm.at[ibuf.at[p, 0]], rows.at[p], sem_g.at[p])

    def _wait_gather(p):
        pltpu.make_async_copy(x_hbm.at[ibuf.at[p, 0]], rows.at[p],
                              sem_g.at[p]).wait()

    gdims = lax.GatherDimensionNumbers(
        offset_dims=(), collapsed_slice_dims=(0,), start_index_map=(0,))

    def _scale(p):
        def _grp(g, carry):
            wreg = wbuf[p, 0, pl.ds(g * LANES, LANES)]
            for e in range(LANES):
                wvec = lax.gather(
                    wreg, jnp.full((LANES, 1), e, jnp.int32), gdims,
                    slice_sizes=(1,),
                    mode=lax.GatherScatterMode.PROMISE_IN_BOUNDS)
                r = g * LANES + e
                for j in range(D_FEAT // LANES):
                    sl = pl.ds(j * LANES, LANES)
                    rows[p, r, sl] = rows[p, r, sl] * wvec
            return carry

        lax.fori_loop(0, BLK // LANES, _grp, 0)

    def _section(b, p, q):
        _wait_gather(p)
        _scale(p)

        @pl.when(b + 1 < NBLK)
        def _():
            _wait_idx(q)
            _issue_gather(q)

        pltpu.sync_copy(rows.at[p], acc.at[ibuf.at[p, 1]], add=True)

        @pl.when(b + 2 < NBLK)
        def _():
            _issue_idx(b + 2, p)

    _issue_idx(0, 0)
    _issue_idx(1, 1)

    def _zacc(i, carry):
        pltpu.sync_copy(rows.at[0],
                        acc.at[pl.ds(s * ROWS_PER_TILE + i * ZROWS, ZROWS)])
        return carry

    lax.fori_loop(0, ROWS_PER_TILE // ZROWS, _zacc, 0)
    _wait_idx(0)
    _issue_gather(0)
    plsc.subcore_barrier()

    def _body(i2, carry):
        _section(2 * i2, 0, 1)
        _section(2 * i2 + 1, 1, 0)
        return carry

    lax.fori_loop(0, NBLK // 2, _body, 0)

    @pl.when(wid < NBLK_EXTRA)
    def _():
        eb = NBLK_TOT - NBLK_EXTRA + wid - blk0
        _issue_idx(eb, 0)
        _wait_idx(0)
        _issue_gather(0)
        _wait_gather(0)
        _scale(0)
        pltpu.sync_copy(rows.at[0], acc.at[ibuf.at[0, 1]], add=True)

    plsc.subcore_barrier()

    pltpu.sync_copy(acc.at[pl.ds(s * ROWS_PER_TILE, ROWS_PER_TILE)],
                    out_hbm.at[c, pl.ds(s * ROWS_PER_TILE, ROWS_PER_TILE)])


_ADD_BS = 512


def _add_body(p_ref, o_ref):
    o_ref[...] = p_ref[0] + p_ref[1]


_combine = pl.pallas_call(
    _add_body,
    grid=(N_PAD // _ADD_BS,),
    in_specs=[pl.BlockSpec((2, _ADD_BS, D_FEAT), lambda i: (0, i, 0))],
    out_specs=pl.BlockSpec((_ADD_BS, D_FEAT), lambda i: (i, 0)),
    out_shape=jax.ShapeDtypeStruct((N_PAD, D_FEAT), jnp.float32),
)


def kernel(features, edge_index, edge_weight, degree):
    row = edge_index[0].astype(jnp.int32).reshape(NBLK_TOT, 1, BLK)
    col = edge_index[1].astype(jnp.int32).reshape(NBLK_TOT, 1, BLK)
    idx2 = jnp.concatenate([col, row], axis=1)
    w3 = edge_weight.astype(jnp.float32).reshape(NBLK_TOT, 1, BLK)
    x0 = jnp.pad(features, ((0, N_PAD - N_NODES), (0, 0)))

    def body(_, x):
        partial = _spmm(x, idx2, w3)
        return _combine(partial)

    out = lax.fori_loop(0, degree, body, x0)
    return out[:N_NODES]
